# baseline (speedup 1.0000x reference)
import functools

import jax
import jax.numpy as jnp
from jax import lax
from jax.experimental import pallas as pl
from jax.experimental.pallas import tpu as pltpu
from jax.experimental.pallas import tpu_sc as plsc

N = 10000
E = 320000
F = 128
NPAD = 10240
D = 8
G4 = 32
NC = 2
NS = 16
NW = NC * NS
EPW = E // NW
EPWP = 10240
CH = 128
NBUF = 8
NCHT = EPWP // CH
NGRP = NCHT // NBUF
RPT = NPAD // NS

_f32 = jnp.float32
_NT = (((1,), (1,)), ((), ()))


@functools.lru_cache(maxsize=1)
def _make_agg():
    mesh = plsc.VectorSubcoreMesh(core_axis_name="c", subcore_axis_name="s",
                                  num_cores=NC, num_subcores=NS)

    @functools.partial(
        pl.kernel,
        out_type=jax.ShapeDtypeStruct((NC, NPAD, D), _f32),
        mesh=mesh,
        compiler_params=pltpu.CompilerParams(use_tc_tiling_on_sc=False),
        scratch_types=[
            pltpu.VMEM((NCHT, CH), jnp.int32),
            pltpu.VMEM((NCHT, CH), jnp.int32),
            pltpu.VMEM((2 * NBUF, CH, D), _f32),
            pltpu.VMEM_SHARED((NPAD, D), _f32),
            pltpu.SemaphoreType.DMA,
            pltpu.SemaphoreType.DMA,
        ],
    )
    def _agg(u_hbm, src_hbm, dst_hbm, out_hbm,
             src_v, dst_v, rows_v, table_sh, gsem, ssem):
        c = lax.axis_index("c")
        s = lax.axis_index("s")
        wid = s * NC + c
        pltpu.sync_copy(src_hbm.at[wid], src_v)
        pltpu.sync_copy(dst_hbm.at[wid], dst_v)
        pltpu.sync_copy(u_hbm.at[pl.ds(s * RPT, RPT)],
                        table_sh.at[pl.ds(s * RPT, RPT)])
        plsc.subcore_barrier()

        def group(g, carry):
            slot_g = lax.rem(g, 2) * NBUF
            slot_p = lax.rem(g + 1, 2) * NBUF
            @pl.when(g >= 2)
            def _():
                for b in range(NBUF):
                    ch = (g - 2) * NBUF + b
                    pltpu.make_async_copy(rows_v.at[slot_g + b],
                                          table_sh.at[dst_v.at[ch]],
                                          ssem).wait()
            @pl.when(g < NGRP)
            def _():
                for b in range(NBUF):
                    ch = g * NBUF + b
                    pltpu.async_copy(u_hbm.at[src_v.at[ch]],
                                     rows_v.at[slot_g + b], gsem)
            @pl.when((g >= 1) & (g <= NGRP))
            def _():
                for b in range(NBUF):
                    ch = (g - 1) * NBUF + b
                    pltpu.make_async_copy(u_hbm.at[src_v.at[ch]],
                                          rows_v.at[slot_p + b], gsem).wait()
                for b in range(NBUF):
                    ch = (g - 1) * NBUF + b
                    pltpu.async_copy(rows_v.at[slot_p + b],
                                     table_sh.at[dst_v.at[ch]], ssem,
                                     add=True)
            return carry

        lax.fori_loop(0, NGRP + 2, group, 0)
        plsc.subcore_barrier()
        pltpu.sync_copy(table_sh.at[pl.ds(s * RPT, RPT)],
                        out_hbm.at[c, pl.ds(s * RPT, RPT)])

    return _agg


def _proj_body(x_ref, w_ref, o_ref):
    o_ref[...] = lax.dot_general(w_ref[...], x_ref[...], _NT,
                                 preferred_element_type=_f32)


_proj = pl.pallas_call(
    _proj_body, out_shape=jax.ShapeDtypeStruct((D, NPAD), _f32))


def _gin_body(u_ref, p0_ref, p1_ref, ba_ref, wb_ref, bb_ref, wna_ref,
              h_ref, v_ref):
    pre = p0_ref[...] + p1_ref[...] - u_ref[...] + ba_ref[...]
    t = jnp.maximum(pre, 0.0)
    h = jnp.dot(wb_ref[...], t, preferred_element_type=_f32) + bb_ref[...]
    h = jnp.maximum(h, 0.0)
    h_ref[...] = h
    v_ref[...] = jnp.dot(wna_ref[...], h, preferred_element_type=_f32)


_gin = pl.pallas_call(
    _gin_body,
    out_shape=(jax.ShapeDtypeStruct((D, NPAD), _f32),
               jax.ShapeDtypeStruct((D, NPAD), _f32)))


def _readout_body(v2_ref, p0_ref, p1_ref, b2a_ref, w2b_ref, b2b_ref,
                  h1_ref, h2_ref,
                  wih_f_ref, whh_f_ref, bias_f_ref,
                  wih_b_ref, whh_b_ref, bias_b_ref,
                  wattf_ref, wattb_ref, batt_ref,
                  wlin_ref, blin_ref, wfc1_ref, bfc1_ref,
                  wfc2_ref, bfc2_ref, o_ref):
    pre = p0_ref[...] + p1_ref[...] - v2_ref[...] + b2a_ref[...]
    t = jnp.maximum(pre, 0.0)
    h3 = jnp.dot(w2b_ref[...], t, preferred_element_type=_f32) + b2b_ref[...]
    h3 = jnp.maximum(h3, 0.0)
    x1, x2, x3 = h1_ref[...], h2_ref[...], h3

    def cell(x_t, h, c, wih, whh, bias):
        g = (jnp.dot(wih, x_t, preferred_element_type=_f32)
             + jnp.dot(whh, h, preferred_element_type=_f32) + bias)
        i = jax.nn.sigmoid(g[0:8])
        f = jax.nn.sigmoid(g[8:16])
        gg = jnp.tanh(g[16:24])
        o = jax.nn.sigmoid(g[24:32])
        c = f * c + i * gg
        return o * jnp.tanh(c), c

    z = jnp.zeros((D, NPAD), _f32)
    wf, hf, bf = wih_f_ref[...], whh_f_ref[...], bias_f_ref[...]
    wb, hb, bb = wih_b_ref[...], whh_b_ref[...], bias_b_ref[...]
    of1, cf = cell(x1, z, z, wf, hf, bf)
    of2, cf = cell(x2, of1, cf, wf, hf, bf)
    of3, cf = cell(x3, of2, cf, wf, hf, bf)
    q1, cb = cell(x3, z, z, wb, hb, bb)
    q2, cb = cell(x2, q1, cb, wb, hb, bb)
    q3, cb = cell(x1, q2, cb, wb, hb, bb)

    wattf, wattb = wattf_ref[...], wattb_ref[...]

    def att(of_t, ob_t):
        return (jnp.sum(of_t * wattf, axis=0, keepdims=True)
                + jnp.sum(ob_t * wattb, axis=0, keepdims=True)
                + batt_ref[...])

    a1, a2, a3 = att(of1, q3), att(of2, q2), att(of3, q1)
    m = jnp.maximum(a1, jnp.maximum(a2, a3))
    e1, e2, e3 = jnp.exp(a1 - m), jnp.exp(a2 - m), jnp.exp(a3 - m)
    ssum = e1 + e2 + e3
    jk = (e1 * x1 + e2 * x2 + e3 * x3) / ssum
    gl = (jnp.dot(wlin_ref[...], jk, preferred_element_type=_f32)
          + blin_ref[...])
    f1 = jnp.sum(gl * wfc1_ref[...], axis=0, keepdims=True) + bfc1_ref[...]
    f1 = jnp.where(f1 >= 0, f1, 0.01 * f1)
    val = jnp.sum(f1 * wfc2_ref[...]) + bfc2_ref[0, 0]
    o_ref[...] = val.reshape(1, 1)


_readout = pl.pallas_call(
    _readout_body, out_shape=jax.ShapeDtypeStruct((1, 1), _f32))


def _padm(w, r, c):
    return jnp.zeros((r, c), _f32).at[: w.shape[0], : w.shape[1]].set(w)


def _padb(b, n=D):
    return jnp.zeros((n, 1), _f32).at[: b.shape[0], 0].set(b)


def _pack_lstm(wih, whh, bih, bhh):
    wi = jnp.zeros((G4, D), _f32)
    wh = jnp.zeros((G4, D), _f32)
    bias = jnp.zeros((G4, 1), _f32)
    bsum = bih + bhh
    for k in range(4):
        wi = wi.at[k * 8 : k * 8 + 7, : wih.shape[1]].set(wih[k * 7 : (k + 1) * 7])
        wh = wh.at[k * 8 : k * 8 + 7, : whh.shape[1]].set(whh[k * 7 : (k + 1) * 7])
        bias = bias.at[k * 8 : k * 8 + 7, 0].set(bsum[k * 7 : (k + 1) * 7])
    return wi, wh, bias


def kernel(x, edge_index, W0a, b0a, W0b, b0b, W1a, b1a, W1b, b1b, W2a, b2a,
           W2b, b2b, lstm_Wih_f, lstm_Whh_f, lstm_bih_f, lstm_bhh_f,
           lstm_Wih_b, lstm_Whh_b, lstm_bih_b, lstm_bhh_b,
           Watt, batt, Wlin, blin, Wfc1, bfc1, Wfc2, bfc2):
    xp = jnp.zeros((NPAD, F), _f32).at[:N].set(x)
    epad = jnp.full((NW, EPWP - EPW), NPAD - 1, jnp.int32)
    src = jnp.concatenate([edge_index[0].reshape(NW, EPW), epad], axis=1)
    src = src.reshape(NW, NCHT, CH)
    dst = jnp.concatenate([edge_index[1].reshape(NW, EPW), epad], axis=1)
    dst = dst.reshape(NW, NCHT, CH)

    w0a_p = _padm(W0a, D, F)
    w0b_p = _padm(W0b, D, D)
    w1a_p = _padm(W1a, D, D)
    w1b_p = _padm(W1b, D, D)
    w2a_p = _padm(W2a, D, D)
    w2b_p = _padm(W2b, D, D)
    b0a_p, b0b_p = _padb(b0a), _padb(b0b)
    b1a_p, b1b_p = _padb(b1a), _padb(b1b)
    b2a_p, b2b_p = _padb(b2a), _padb(b2b)
    wih_f, whh_f, bias_f = _pack_lstm(lstm_Wih_f, lstm_Whh_f,
                                      lstm_bih_f, lstm_bhh_f)
    wih_b, whh_b, bias_b = _pack_lstm(lstm_Wih_b, lstm_Whh_b,
                                      lstm_bih_b, lstm_bhh_b)
    wattf = _padb(Watt[0, :7])
    wattb = _padb(Watt[0, 7:14])
    batt_p = batt.reshape(1, 1)
    wlin_p = _padm(Wlin, D, D)
    blin_p = _padb(blin)
    wfc1_p = _padb(Wfc1[0])
    bfc1_p = bfc1.reshape(1, 1)
    wfc2_p = jnp.zeros((1, NPAD), _f32).at[0, :N].set(Wfc2[0])
    bfc2_p = bfc2.reshape(1, 1)

    agg = _make_agg()
    u0T = _proj(xp, w0a_p)
    p0 = agg(u0T.T, src, dst)
    h1T, v1T = _gin(u0T, p0[0].T, p0[1].T, b0a_p, w0b_p, b0b_p, w1a_p)
    p1 = agg(v1T.T, src, dst)
    h2T, v2T = _gin(v1T, p1[0].T, p1[1].T, b1a_p, w1b_p, b1b_p, w2a_p)
    p2 = agg(v2T.T, src, dst)
    res = _readout(v2T, p2[0].T, p2[1].T, b2a_p, w2b_p, b2b_p, h1T, h2T,
                   wih_f, whh_f, bias_f, wih_b, whh_b, bias_b,
                   wattf, wattb, batt_p, wlin_p, blin_p, wfc1_p, bfc1_p,
                   wfc2_p, bfc2_p)
    return res[0]


# speedup vs baseline: 1.8508x; 1.8508x over previous
import functools

import jax
import jax.numpy as jnp
from jax import lax
from jax.experimental import pallas as pl
from jax.experimental.pallas import tpu as pltpu
from jax.experimental.pallas import tpu_sc as plsc

N = 10000
E = 320000
F = 128
NPAD = 10240
D = 8
G4 = 32
NC = 2
NS = 16
NW = NC * NS
EPW = E // NW
CH = 80
NBUF = 5
NCHT = EPW // CH
NGRP = NCHT // NBUF
RPT = NPAD // NS

_f32 = jnp.float32
_NT = (((1,), (1,)), ((), ()))


@functools.lru_cache(maxsize=1)
def _make_agg():
    mesh = plsc.VectorSubcoreMesh(core_axis_name="c", subcore_axis_name="s",
                                  num_cores=NC, num_subcores=NS)

    @functools.partial(
        pl.kernel,
        out_type=(jax.ShapeDtypeStruct((NC, D, NPAD), _f32),
                  jax.ShapeDtypeStruct((NC, NPAD, D), _f32)),
        mesh=mesh,
        compiler_params=pltpu.CompilerParams(use_tc_tiling_on_sc=False,
                                             needs_layout_passes=False),
        scratch_types=[
            pltpu.VMEM((NCHT, CH), jnp.int32),
            pltpu.VMEM((NCHT, CH), jnp.int32),
            pltpu.VMEM((2 * NBUF, CH, D), _f32),
            pltpu.VMEM_SHARED((NPAD, D), _f32),
            pltpu.VMEM((D, RPT), _f32),
            pltpu.VMEM((RPT, D), _f32),
            pltpu.SemaphoreType.DMA,
            pltpu.SemaphoreType.DMA,
        ],
    )
    def _agg(ut_hbm, src_hbm, dst_hbm, pt_hbm, unm_hbm,
             src_v, dst_v, rows_v, table_sh, fm_v, nm_v, gsem, ssem):
        c = lax.axis_index("c")
        s = lax.axis_index("s")
        wid = s * NC + c
        lanes = lax.iota(jnp.int32, 16)
        pltpu.sync_copy(src_hbm.at[wid], src_v)
        pltpu.sync_copy(dst_hbm.at[wid], dst_v)
        for f in range(D):
            pltpu.sync_copy(ut_hbm.at[f, pl.ds(s * RPT, RPT)], fm_v.at[f])

        def t_fwd(grp, carry):
            base = grp * 16
            ridx = base + lanes
            for f in range(D):
                vals = fm_v[f, pl.ds(base, 16)]
                plsc.store_scatter(nm_v, [ridx, jnp.full((16,), f, jnp.int32)],
                                   vals)
            return carry

        lax.fori_loop(0, RPT // 16, t_fwd, 0)
        pltpu.sync_copy(nm_v, unm_hbm.at[c, pl.ds(s * RPT, RPT)])
        pltpu.sync_copy(nm_v, table_sh.at[pl.ds(s * RPT, RPT)])
        plsc.subcore_barrier()
        u_hbm = unm_hbm.at[c]

        def group(g, carry):
            slot_g = lax.rem(g, 2) * NBUF
            slot_p = lax.rem(g + 1, 2) * NBUF
            @pl.when(g >= 2)
            def _():
                for b in range(NBUF):
                    ch = (g - 2) * NBUF + b
                    pltpu.make_async_copy(rows_v.at[slot_g + b],
                                          table_sh.at[dst_v.at[ch]],
                                          ssem).wait()
            @pl.when(g < NGRP)
            def _():
                for b in range(NBUF):
                    ch = g * NBUF + b
                    pltpu.async_copy(u_hbm.at[src_v.at[ch]],
                                     rows_v.at[slot_g + b], gsem)
            @pl.when((g >= 1) & (g <= NGRP))
            def _():
                for b in range(NBUF):
                    ch = (g - 1) * NBUF + b
                    pltpu.make_async_copy(u_hbm.at[src_v.at[ch]],
                                          rows_v.at[slot_p + b], gsem).wait()
                for b in range(NBUF):
                    ch = (g - 1) * NBUF + b
                    pltpu.async_copy(rows_v.at[slot_p + b],
                                     table_sh.at[dst_v.at[ch]], ssem,
                                     add=True)
            return carry

        lax.fori_loop(0, NGRP + 2, group, 0)
        plsc.subcore_barrier()
        pltpu.sync_copy(table_sh.at[pl.ds(s * RPT, RPT)], nm_v)

        def t_bwd(grp, carry):
            base = grp * 16
            ridx = base + lanes
            for f in range(D):
                vals = plsc.load_gather(
                    nm_v, [ridx, jnp.full((16,), f, jnp.int32)])
                fm_v[f, pl.ds(base, 16)] = vals
            return carry

        lax.fori_loop(0, RPT // 16, t_bwd, 0)
        for f in range(D):
            pltpu.sync_copy(fm_v.at[f], pt_hbm.at[c, f, pl.ds(s * RPT, RPT)])

    return _agg


def _proj_body(x_ref, w_ref, o_ref):
    o_ref[...] = lax.dot_general(w_ref[...], x_ref[...], _NT,
                                 preferred_element_type=_f32)


_proj = pl.pallas_call(
    _proj_body, out_shape=jax.ShapeDtypeStruct((D, NPAD), _f32))


def _gin_body(u_ref, p0_ref, p1_ref, ba_ref, wb_ref, bb_ref, wna_ref,
              h_ref, v_ref):
    pre = p0_ref[...] + p1_ref[...] - u_ref[...] + ba_ref[...]
    t = jnp.maximum(pre, 0.0)
    h = jnp.dot(wb_ref[...], t, preferred_element_type=_f32) + bb_ref[...]
    h = jnp.maximum(h, 0.0)
    h_ref[...] = h
    v_ref[...] = jnp.dot(wna_ref[...], h, preferred_element_type=_f32)


_gin = pl.pallas_call(
    _gin_body,
    out_shape=(jax.ShapeDtypeStruct((D, NPAD), _f32),
               jax.ShapeDtypeStruct((D, NPAD), _f32)))


def _readout_body(v2_ref, p0_ref, p1_ref, b2a_ref, w2b_ref, b2b_ref,
                  h1_ref, h2_ref,
                  wih_f_ref, whh_f_ref, bias_f_ref,
                  wih_b_ref, whh_b_ref, bias_b_ref,
                  wattf_ref, wattb_ref, batt_ref,
                  wlin_ref, blin_ref, wfc1_ref, bfc1_ref,
                  wfc2_ref, bfc2_ref, o_ref):
    pre = p0_ref[...] + p1_ref[...] - v2_ref[...] + b2a_ref[...]
    t = jnp.maximum(pre, 0.0)
    h3 = jnp.dot(w2b_ref[...], t, preferred_element_type=_f32) + b2b_ref[...]
    h3 = jnp.maximum(h3, 0.0)
    x1, x2, x3 = h1_ref[...], h2_ref[...], h3

    def cell(x_t, h, c, wih, whh, bias):
        g = (jnp.dot(wih, x_t, preferred_element_type=_f32)
             + jnp.dot(whh, h, preferred_element_type=_f32) + bias)
        i = jax.nn.sigmoid(g[0:8])
        f = jax.nn.sigmoid(g[8:16])
        gg = jnp.tanh(g[16:24])
        o = jax.nn.sigmoid(g[24:32])
        c = f * c + i * gg
        return o * jnp.tanh(c), c

    z = jnp.zeros((D, NPAD), _f32)
    wf, hf, bf = wih_f_ref[...], whh_f_ref[...], bias_f_ref[...]
    wb, hb, bb = wih_b_ref[...], whh_b_ref[...], bias_b_ref[...]
    of1, cf = cell(x1, z, z, wf, hf, bf)
    of2, cf = cell(x2, of1, cf, wf, hf, bf)
    of3, cf = cell(x3, of2, cf, wf, hf, bf)
    q1, cb = cell(x3, z, z, wb, hb, bb)
    q2, cb = cell(x2, q1, cb, wb, hb, bb)
    q3, cb = cell(x1, q2, cb, wb, hb, bb)

    wattf, wattb = wattf_ref[...], wattb_ref[...]

    def att(of_t, ob_t):
        return (jnp.sum(of_t * wattf, axis=0, keepdims=True)
                + jnp.sum(ob_t * wattb, axis=0, keepdims=True)
                + batt_ref[...])

    a1, a2, a3 = att(of1, q3), att(of2, q2), att(of3, q1)
    m = jnp.maximum(a1, jnp.maximum(a2, a3))
    e1, e2, e3 = jnp.exp(a1 - m), jnp.exp(a2 - m), jnp.exp(a3 - m)
    ssum = e1 + e2 + e3
    jk = (e1 * x1 + e2 * x2 + e3 * x3) / ssum
    gl = (jnp.dot(wlin_ref[...], jk, preferred_element_type=_f32)
          + blin_ref[...])
    f1 = jnp.sum(gl * wfc1_ref[...], axis=0, keepdims=True) + bfc1_ref[...]
    f1 = jnp.where(f1 >= 0, f1, 0.01 * f1)
    val = jnp.sum(f1 * wfc2_ref[...]) + bfc2_ref[0, 0]
    o_ref[...] = val.reshape(1, 1)


_readout = pl.pallas_call(
    _readout_body, out_shape=jax.ShapeDtypeStruct((1, 1), _f32))


def _padm(w, r, c):
    return jnp.zeros((r, c), _f32).at[: w.shape[0], : w.shape[1]].set(w)


def _padb(b, n=D):
    return jnp.zeros((n, 1), _f32).at[: b.shape[0], 0].set(b)


def _pack_lstm(wih, whh, bih, bhh):
    wi = jnp.zeros((G4, D), _f32)
    wh = jnp.zeros((G4, D), _f32)
    bias = jnp.zeros((G4, 1), _f32)
    bsum = bih + bhh
    for k in range(4):
        wi = wi.at[k * 8 : k * 8 + 7, : wih.shape[1]].set(wih[k * 7 : (k + 1) * 7])
        wh = wh.at[k * 8 : k * 8 + 7, : whh.shape[1]].set(whh[k * 7 : (k + 1) * 7])
        bias = bias.at[k * 8 : k * 8 + 7, 0].set(bsum[k * 7 : (k + 1) * 7])
    return wi, wh, bias


def kernel(x, edge_index, W0a, b0a, W0b, b0b, W1a, b1a, W1b, b1b, W2a, b2a,
           W2b, b2b, lstm_Wih_f, lstm_Whh_f, lstm_bih_f, lstm_bhh_f,
           lstm_Wih_b, lstm_Whh_b, lstm_bih_b, lstm_bhh_b,
           Watt, batt, Wlin, blin, Wfc1, bfc1, Wfc2, bfc2):
    xp = jnp.zeros((NPAD, F), _f32).at[:N].set(x)
    src = edge_index[0].reshape(NW, NCHT, CH)
    dst = edge_index[1].reshape(NW, NCHT, CH)

    w0a_p = _padm(W0a, D, F)
    w0b_p = _padm(W0b, D, D)
    w1a_p = _padm(W1a, D, D)
    w1b_p = _padm(W1b, D, D)
    w2a_p = _padm(W2a, D, D)
    w2b_p = _padm(W2b, D, D)
    b0a_p, b0b_p = _padb(b0a), _padb(b0b)
    b1a_p, b1b_p = _padb(b1a), _padb(b1b)
    b2a_p, b2b_p = _padb(b2a), _padb(b2b)
    wih_f, whh_f, bias_f = _pack_lstm(lstm_Wih_f, lstm_Whh_f,
                                      lstm_bih_f, lstm_bhh_f)
    wih_b, whh_b, bias_b = _pack_lstm(lstm_Wih_b, lstm_Whh_b,
                                      lstm_bih_b, lstm_bhh_b)
    wattf = _padb(Watt[0, :7])
    wattb = _padb(Watt[0, 7:14])
    batt_p = batt.reshape(1, 1)
    wlin_p = _padm(Wlin, D, D)
    blin_p = _padb(blin)
    wfc1_p = _padb(Wfc1[0])
    bfc1_p = bfc1.reshape(1, 1)
    wfc2_p = jnp.zeros((1, NPAD), _f32).at[0, :N].set(Wfc2[0])
    bfc2_p = bfc2.reshape(1, 1)

    agg = _make_agg()
    u0T = _proj(xp, w0a_p)
    p0, _ = agg(u0T, src, dst)
    h1T, v1T = _gin(u0T, p0[0], p0[1], b0a_p, w0b_p, b0b_p, w1a_p)
    p1, _ = agg(v1T, src, dst)
    h2T, v2T = _gin(v1T, p1[0], p1[1], b1a_p, w1b_p, b1b_p, w2a_p)
    p2, _ = agg(v2T, src, dst)
    res = _readout(v2T, p2[0], p2[1], b2a_p, w2b_p, b2b_p, h1T, h2T,
                   wih_f, whh_f, bias_f, wih_b, whh_b, bias_b,
                   wattf, wattb, batt_p, wlin_p, blin_p, wfc1_p, bfc1_p,
                   wfc2_p, bfc2_p)
    return res[0]


# speedup vs baseline: 2.2114x; 1.1949x over previous
import functools

import jax
import jax.numpy as jnp
from jax import lax
from jax.experimental import pallas as pl
from jax.experimental.pallas import tpu as pltpu
from jax.experimental.pallas import tpu_sc as plsc

N = 10000
E = 320000
F = 128
NPAD = 10240
D = 8
G4 = 32
NC = 2
NS = 16
NW = NC * NS
EPW = E // NW
CH = 80
NBUF = 5
NCHT = EPW // CH
NGRP = NCHT // NBUF
RPT = NPAD // NS

_f32 = jnp.float32
_NT = (((1,), (1,)), ((), ()))


@functools.lru_cache(maxsize=1)
def _make_agg():
    mesh = plsc.VectorSubcoreMesh(core_axis_name="c", subcore_axis_name="s",
                                  num_cores=NC, num_subcores=NS)

    @functools.partial(
        pl.kernel,
        out_type=(jax.ShapeDtypeStruct((NC, D, NPAD), _f32),
                  jax.ShapeDtypeStruct((NC, NPAD, D), _f32)),
        mesh=mesh,
        compiler_params=pltpu.CompilerParams(use_tc_tiling_on_sc=False,
                                             needs_layout_passes=False),
        scratch_types=[
            pltpu.VMEM((NCHT, CH), jnp.int32),
            pltpu.VMEM((NCHT, CH), jnp.int32),
            pltpu.VMEM((2 * NBUF, CH, D), _f32),
            pltpu.VMEM_SHARED((NPAD, D), _f32),
            pltpu.VMEM((D, RPT), _f32),
            pltpu.VMEM((RPT, D), _f32),
            pltpu.SemaphoreType.DMA,
            pltpu.SemaphoreType.DMA,
        ],
    )
    def _agg(ut_hbm, e_hbm, pt_hbm, unm_hbm,
             src_v, dst_v, rows_v, table_sh, fm_v, nm_v, gsem, ssem):
        c = lax.axis_index("c")
        s = lax.axis_index("s")
        wid = s * NC + c
        lanes = lax.iota(jnp.int32, 16)
        pltpu.async_copy(e_hbm.at[0, wid], src_v, ssem)
        pltpu.async_copy(e_hbm.at[1, wid], dst_v, ssem)
        for f in range(D):
            pltpu.async_copy(ut_hbm.at[f, pl.ds(s * RPT, RPT)], fm_v.at[f],
                             gsem)
        for f in range(D):
            pltpu.make_async_copy(ut_hbm.at[f, pl.ds(s * RPT, RPT)],
                                  fm_v.at[f], gsem).wait()
        pltpu.make_async_copy(e_hbm.at[0, wid], src_v, ssem).wait()
        pltpu.make_async_copy(e_hbm.at[1, wid], dst_v, ssem).wait()

        def t_fwd(grp, carry):
            base = grp * 16
            ridx = base + lanes
            for f in range(D):
                vals = fm_v[f, pl.ds(base, 16)]
                plsc.store_scatter(nm_v, [ridx, jnp.full((16,), f, jnp.int32)],
                                   vals)
            return carry

        lax.fori_loop(0, RPT // 16, t_fwd, 0)
        pltpu.sync_copy(nm_v, unm_hbm.at[c, pl.ds(s * RPT, RPT)])
        pltpu.sync_copy(nm_v, table_sh.at[pl.ds(s * RPT, RPT)])
        plsc.subcore_barrier()
        u_hbm = unm_hbm.at[c]

        def group(g, carry):
            slot_g = lax.rem(g, 2) * NBUF
            slot_p = lax.rem(g + 1, 2) * NBUF
            @pl.when(g >= 2)
            def _():
                for b in range(NBUF):
                    ch = (g - 2) * NBUF + b
                    pltpu.make_async_copy(rows_v.at[slot_g + b],
                                          table_sh.at[dst_v.at[ch]],
                                          ssem).wait()
            @pl.when(g < NGRP)
            def _():
                for b in range(NBUF):
                    ch = g * NBUF + b
                    pltpu.async_copy(u_hbm.at[src_v.at[ch]],
                                     rows_v.at[slot_g + b], gsem)
            @pl.when((g >= 1) & (g <= NGRP))
            def _():
                for b in range(NBUF):
                    ch = (g - 1) * NBUF + b
                    pltpu.make_async_copy(u_hbm.at[src_v.at[ch]],
                                          rows_v.at[slot_p + b], gsem).wait()
                for b in range(NBUF):
                    ch = (g - 1) * NBUF + b
                    pltpu.async_copy(rows_v.at[slot_p + b],
                                     table_sh.at[dst_v.at[ch]], ssem,
                                     add=True)
            return carry

        lax.fori_loop(0, NGRP + 2, group, 0)
        plsc.subcore_barrier()
        pltpu.sync_copy(table_sh.at[pl.ds(s * RPT, RPT)], nm_v)

        def t_bwd(grp, carry):
            base = grp * 16
            ridx = base + lanes
            for f in range(D):
                vals = plsc.load_gather(
                    nm_v, [ridx, jnp.full((16,), f, jnp.int32)])
                fm_v[f, pl.ds(base, 16)] = vals
            return carry

        lax.fori_loop(0, RPT // 16, t_bwd, 0)
        for f in range(D):
            pltpu.async_copy(fm_v.at[f], pt_hbm.at[c, f, pl.ds(s * RPT, RPT)],
                             gsem)
        for f in range(D):
            pltpu.make_async_copy(fm_v.at[f],
                                  pt_hbm.at[c, f, pl.ds(s * RPT, RPT)],
                                  gsem).wait()

    return _agg


def _proj_body(x_ref, w_ref, o_ref):
    o_ref[...] = lax.dot_general(w_ref[...], x_ref[...], _NT,
                                 preferred_element_type=_f32)


_proj = pl.pallas_call(
    _proj_body, out_shape=jax.ShapeDtypeStruct((D, N), _f32))


def _gin_body(u_ref, p0_ref, p1_ref, ba_ref, wb_ref, bb_ref, wna_ref,
              h_ref, v_ref):
    pre = p0_ref[...] + p1_ref[...] - u_ref[...] + ba_ref[...]
    t = jnp.maximum(pre, 0.0)
    h = jnp.dot(wb_ref[...], t, preferred_element_type=_f32) + bb_ref[...]
    h = jnp.maximum(h, 0.0)
    h_ref[...] = h
    v_ref[...] = jnp.dot(wna_ref[...], h, preferred_element_type=_f32)


_gin = pl.pallas_call(
    _gin_body,
    out_shape=(jax.ShapeDtypeStruct((D, NPAD), _f32),
               jax.ShapeDtypeStruct((D, NPAD), _f32)))


def _readout_body(v2_ref, p0_ref, p1_ref, b2a_ref, w2b_ref, b2b_ref,
                  h1_ref, h2_ref,
                  wih_f_ref, whh_f_ref, bias_f_ref,
                  wih_b_ref, whh_b_ref, bias_b_ref,
                  wattf_ref, wattb_ref, batt_ref,
                  wlin_ref, blin_ref, wfc1_ref, bfc1_ref,
                  wfc2_ref, bfc2_ref, o_ref):
    pre = p0_ref[...] + p1_ref[...] - v2_ref[...] + b2a_ref[...]
    t = jnp.maximum(pre, 0.0)
    h3 = jnp.dot(w2b_ref[...], t, preferred_element_type=_f32) + b2b_ref[...]
    h3 = jnp.maximum(h3, 0.0)
    x1, x2, x3 = h1_ref[...], h2_ref[...], h3

    def cell(x_t, h, c, wih, whh, bias):
        g = (jnp.dot(wih, x_t, preferred_element_type=_f32)
             + jnp.dot(whh, h, preferred_element_type=_f32) + bias)
        i = jax.nn.sigmoid(g[0:8])
        f = jax.nn.sigmoid(g[8:16])
        gg = jnp.tanh(g[16:24])
        o = jax.nn.sigmoid(g[24:32])
        c = f * c + i * gg
        return o * jnp.tanh(c), c

    z = jnp.zeros((D, NPAD), _f32)
    wf, hf, bf = wih_f_ref[...], whh_f_ref[...], bias_f_ref[...]
    wb, hb, bb = wih_b_ref[...], whh_b_ref[...], bias_b_ref[...]
    of1, cf = cell(x1, z, z, wf, hf, bf)
    of2, cf = cell(x2, of1, cf, wf, hf, bf)
    of3, cf = cell(x3, of2, cf, wf, hf, bf)
    q1, cb = cell(x3, z, z, wb, hb, bb)
    q2, cb = cell(x2, q1, cb, wb, hb, bb)
    q3, cb = cell(x1, q2, cb, wb, hb, bb)

    wattf, wattb = wattf_ref[...], wattb_ref[...]

    def att(of_t, ob_t):
        return (jnp.sum(of_t * wattf, axis=0, keepdims=True)
                + jnp.sum(ob_t * wattb, axis=0, keepdims=True)
                + batt_ref[...])

    a1, a2, a3 = att(of1, q3), att(of2, q2), att(of3, q1)
    m = jnp.maximum(a1, jnp.maximum(a2, a3))
    e1, e2, e3 = jnp.exp(a1 - m), jnp.exp(a2 - m), jnp.exp(a3 - m)
    ssum = e1 + e2 + e3
    jk = (e1 * x1 + e2 * x2 + e3 * x3) / ssum
    gl = (jnp.dot(wlin_ref[...], jk, preferred_element_type=_f32)
          + blin_ref[...])
    f1 = jnp.sum(gl * wfc1_ref[...], axis=0, keepdims=True) + bfc1_ref[...]
    f1 = jnp.where(f1 >= 0, f1, 0.01 * f1)
    val = jnp.sum(f1 * wfc2_ref[...]) + bfc2_ref[0, 0]
    o_ref[...] = val.reshape(1, 1)


_readout = pl.pallas_call(
    _readout_body, out_shape=jax.ShapeDtypeStruct((1, 1), _f32))


def _padm(w, r, c):
    return jnp.zeros((r, c), _f32).at[: w.shape[0], : w.shape[1]].set(w)


def _padb(b, n=D):
    return jnp.zeros((n, 1), _f32).at[: b.shape[0], 0].set(b)


def _pack_lstm(wih, whh, bih, bhh):
    wi = jnp.zeros((G4, D), _f32)
    wh = jnp.zeros((G4, D), _f32)
    bias = jnp.zeros((G4, 1), _f32)
    bsum = bih + bhh
    for k in range(4):
        wi = wi.at[k * 8 : k * 8 + 7, : wih.shape[1]].set(wih[k * 7 : (k + 1) * 7])
        wh = wh.at[k * 8 : k * 8 + 7, : whh.shape[1]].set(whh[k * 7 : (k + 1) * 7])
        bias = bias.at[k * 8 : k * 8 + 7, 0].set(bsum[k * 7 : (k + 1) * 7])
    return wi, wh, bias


def kernel(x, edge_index, W0a, b0a, W0b, b0b, W1a, b1a, W1b, b1b, W2a, b2a,
           W2b, b2b, lstm_Wih_f, lstm_Whh_f, lstm_bih_f, lstm_bhh_f,
           lstm_Wih_b, lstm_Whh_b, lstm_bih_b, lstm_bhh_b,
           Watt, batt, Wlin, blin, Wfc1, bfc1, Wfc2, bfc2):
    er = edge_index.reshape(2, NW, NCHT, CH)

    w0a_p = _padm(W0a, D, F)
    w0b_p = _padm(W0b, D, D)
    w1a_p = _padm(W1a, D, D)
    w1b_p = _padm(W1b, D, D)
    w2a_p = _padm(W2a, D, D)
    w2b_p = _padm(W2b, D, D)
    b0a_p, b0b_p = _padb(b0a), _padb(b0b)
    b1a_p, b1b_p = _padb(b1a), _padb(b1b)
    b2a_p, b2b_p = _padb(b2a), _padb(b2b)
    wih_f, whh_f, bias_f = _pack_lstm(lstm_Wih_f, lstm_Whh_f,
                                      lstm_bih_f, lstm_bhh_f)
    wih_b, whh_b, bias_b = _pack_lstm(lstm_Wih_b, lstm_Whh_b,
                                      lstm_bih_b, lstm_bhh_b)
    wattf = _padb(Watt[0, :7])
    wattb = _padb(Watt[0, 7:14])
    batt_p = batt.reshape(1, 1)
    wlin_p = _padm(Wlin, D, D)
    blin_p = _padb(blin)
    wfc1_p = _padb(Wfc1[0])
    bfc1_p = bfc1.reshape(1, 1)
    wfc2_p = jnp.zeros((1, NPAD), _f32).at[0, :N].set(Wfc2[0])
    bfc2_p = bfc2.reshape(1, 1)

    agg = _make_agg()
    u0T = jnp.pad(_proj(x, w0a_p), ((0, 0), (0, NPAD - N)))
    p0, _ = agg(u0T, er)
    h1T, v1T = _gin(u0T, p0[0], p0[1], b0a_p, w0b_p, b0b_p, w1a_p)
    p1, _ = agg(v1T, er)
    h2T, v2T = _gin(v1T, p1[0], p1[1], b1a_p, w1b_p, b1b_p, w2a_p)
    p2, _ = agg(v2T, er)
    res = _readout(v2T, p2[0], p2[1], b2a_p, w2b_p, b2b_p, h1T, h2T,
                   wih_f, whh_f, bias_f, wih_b, whh_b, bias_b,
                   wattf, wattb, batt_p, wlin_p, blin_p, wfc1_p, bfc1_p,
                   wfc2_p, bfc2_p)
    return res[0]


# speedup vs baseline: 3.1216x; 1.4116x over previous
import functools

import jax
import jax.numpy as jnp
from jax import lax
from jax.experimental import pallas as pl
from jax.experimental.pallas import tpu as pltpu
from jax.experimental.pallas import tpu_sc as plsc

N = 10000
E = 320000
F = 128
NPAD = 10240
D = 8
G4 = 32
NC = 2
NS = 16
NW = NC * NS
EPW = E // NW
CH = 80
NBUF = 5
NCHT = EPW // CH
NGRP = NCHT // NBUF
RPT = NPAD // NS

_f32 = jnp.float32
_NT = (((1,), (1,)), ((), ()))


@functools.lru_cache(maxsize=1)
def _make_agg():
    mesh = plsc.VectorSubcoreMesh(core_axis_name="c", subcore_axis_name="s",
                                  num_cores=NC, num_subcores=NS)

    @functools.partial(
        pl.kernel,
        out_type=jax.ShapeDtypeStruct((NC, D, NPAD), _f32),
        mesh=mesh,
        compiler_params=pltpu.CompilerParams(use_tc_tiling_on_sc=False,
                                             needs_layout_passes=False),
        scratch_types=[
            pltpu.VMEM((NCHT, CH), jnp.int32),
            pltpu.VMEM((NCHT, CH), jnp.int32),
            pltpu.VMEM((2 * NBUF, CH, D), _f32),
            pltpu.VMEM_SHARED((NPAD, D), _f32),
            pltpu.VMEM_SHARED((NPAD, D), _f32),
            pltpu.VMEM((D, RPT), _f32),
            pltpu.VMEM((RPT, D), _f32),
            pltpu.SemaphoreType.DMA,
            pltpu.SemaphoreType.DMA,
        ],
    )
    def _agg(ut_hbm, e_hbm, pt_hbm,
             src_v, dst_v, rows_v, table_sh, u_sh, fm_v, nm_v, gsem, ssem):
        c = lax.axis_index("c")
        s = lax.axis_index("s")
        wid = s * NC + c
        lanes = lax.iota(jnp.int32, 16)
        pltpu.async_copy(e_hbm.at[0, wid], src_v, ssem)
        pltpu.async_copy(e_hbm.at[1, wid], dst_v, ssem)
        for f in range(D):
            pltpu.async_copy(ut_hbm.at[f, pl.ds(s * RPT, RPT)], fm_v.at[f],
                             gsem)
        for f in range(D):
            pltpu.make_async_copy(ut_hbm.at[f, pl.ds(s * RPT, RPT)],
                                  fm_v.at[f], gsem).wait()
        pltpu.make_async_copy(e_hbm.at[0, wid], src_v, ssem).wait()
        pltpu.make_async_copy(e_hbm.at[1, wid], dst_v, ssem).wait()

        def t_fwd(grp, carry):
            base = grp * 16
            ridx = base + lanes
            for f in range(D):
                vals = fm_v[f, pl.ds(base, 16)]
                plsc.store_scatter(nm_v, [ridx, jnp.full((16,), f, jnp.int32)],
                                   vals)
            return carry

        lax.fori_loop(0, RPT // 16, t_fwd, 0)
        pltpu.sync_copy(nm_v, u_sh.at[pl.ds(s * RPT, RPT)])
        pltpu.sync_copy(nm_v, table_sh.at[pl.ds(s * RPT, RPT)])
        plsc.subcore_barrier()
        u_hbm = u_sh

        def group(g, carry):
            slot_g = lax.rem(g, 2) * NBUF
            slot_p = lax.rem(g + 1, 2) * NBUF
            @pl.when(g >= 2)
            def _():
                for b in range(NBUF):
                    ch = (g - 2) * NBUF + b
                    pltpu.make_async_copy(rows_v.at[slot_g + b],
                                          table_sh.at[dst_v.at[ch]],
                                          ssem).wait()
            @pl.when(g < NGRP)
            def _():
                for b in range(NBUF):
                    ch = g * NBUF + b
                    pltpu.async_copy(u_hbm.at[src_v.at[ch]],
                                     rows_v.at[slot_g + b], gsem)
            @pl.when((g >= 1) & (g <= NGRP))
            def _():
                for b in range(NBUF):
                    ch = (g - 1) * NBUF + b
                    pltpu.make_async_copy(u_hbm.at[src_v.at[ch]],
                                          rows_v.at[slot_p + b], gsem).wait()
                for b in range(NBUF):
                    ch = (g - 1) * NBUF + b
                    pltpu.async_copy(rows_v.at[slot_p + b],
                                     table_sh.at[dst_v.at[ch]], ssem,
                                     add=True)
            return carry

        lax.fori_loop(0, NGRP + 2, group, 0)
        plsc.subcore_barrier()
        pltpu.sync_copy(table_sh.at[pl.ds(s * RPT, RPT)], nm_v)

        def t_bwd(grp, carry):
            base = grp * 16
            ridx = base + lanes
            for f in range(D):
                vals = plsc.load_gather(
                    nm_v, [ridx, jnp.full((16,), f, jnp.int32)])
                fm_v[f, pl.ds(base, 16)] = vals
            return carry

        lax.fori_loop(0, RPT // 16, t_bwd, 0)
        for f in range(D):
            pltpu.async_copy(fm_v.at[f], pt_hbm.at[c, f, pl.ds(s * RPT, RPT)],
                             gsem)
        for f in range(D):
            pltpu.make_async_copy(fm_v.at[f],
                                  pt_hbm.at[c, f, pl.ds(s * RPT, RPT)],
                                  gsem).wait()

    return _agg


def _proj_body(x_ref, w_ref, o_ref):
    o_ref[...] = lax.dot_general(w_ref[...], x_ref[...], _NT,
                                 preferred_element_type=_f32)


_proj = pl.pallas_call(
    _proj_body, out_shape=jax.ShapeDtypeStruct((D, N), _f32))


def _gin_body(u_ref, p0_ref, p1_ref, ba_ref, wb_ref, bb_ref, wna_ref,
              h_ref, v_ref):
    pre = p0_ref[...] + p1_ref[...] - u_ref[...] + ba_ref[...]
    t = jnp.maximum(pre, 0.0)
    h = jnp.dot(wb_ref[...], t, preferred_element_type=_f32) + bb_ref[...]
    h = jnp.maximum(h, 0.0)
    h_ref[...] = h
    v_ref[...] = jnp.dot(wna_ref[...], h, preferred_element_type=_f32)


_gin = pl.pallas_call(
    _gin_body,
    out_shape=(jax.ShapeDtypeStruct((D, NPAD), _f32),
               jax.ShapeDtypeStruct((D, NPAD), _f32)))


def _readout_body(v2_ref, p0_ref, p1_ref, b2a_ref, w2b_ref, b2b_ref,
                  h1_ref, h2_ref,
                  wih_f_ref, whh_f_ref, bias_f_ref,
                  wih_b_ref, whh_b_ref, bias_b_ref,
                  wattf_ref, wattb_ref, batt_ref,
                  wlin_ref, blin_ref, wfc1_ref, bfc1_ref,
                  wfc2_ref, bfc2_ref, o_ref):
    pre = p0_ref[...] + p1_ref[...] - v2_ref[...] + b2a_ref[...]
    t = jnp.maximum(pre, 0.0)
    h3 = jnp.dot(w2b_ref[...], t, preferred_element_type=_f32) + b2b_ref[...]
    h3 = jnp.maximum(h3, 0.0)
    x1, x2, x3 = h1_ref[...], h2_ref[...], h3

    def cell(x_t, h, c, wih, whh, bias):
        g = (jnp.dot(wih, x_t, preferred_element_type=_f32)
             + jnp.dot(whh, h, preferred_element_type=_f32) + bias)
        i = jax.nn.sigmoid(g[0:8])
        f = jax.nn.sigmoid(g[8:16])
        gg = jnp.tanh(g[16:24])
        o = jax.nn.sigmoid(g[24:32])
        c = f * c + i * gg
        return o * jnp.tanh(c), c

    z = jnp.zeros((D, NPAD), _f32)
    wf, hf, bf = wih_f_ref[...], whh_f_ref[...], bias_f_ref[...]
    wb, hb, bb = wih_b_ref[...], whh_b_ref[...], bias_b_ref[...]
    of1, cf = cell(x1, z, z, wf, hf, bf)
    of2, cf = cell(x2, of1, cf, wf, hf, bf)
    of3, cf = cell(x3, of2, cf, wf, hf, bf)
    q1, cb = cell(x3, z, z, wb, hb, bb)
    q2, cb = cell(x2, q1, cb, wb, hb, bb)
    q3, cb = cell(x1, q2, cb, wb, hb, bb)

    wattf, wattb = wattf_ref[...], wattb_ref[...]

    def att(of_t, ob_t):
        return (jnp.sum(of_t * wattf, axis=0, keepdims=True)
                + jnp.sum(ob_t * wattb, axis=0, keepdims=True)
                + batt_ref[...])

    a1, a2, a3 = att(of1, q3), att(of2, q2), att(of3, q1)
    m = jnp.maximum(a1, jnp.maximum(a2, a3))
    e1, e2, e3 = jnp.exp(a1 - m), jnp.exp(a2 - m), jnp.exp(a3 - m)
    ssum = e1 + e2 + e3
    jk = (e1 * x1 + e2 * x2 + e3 * x3) / ssum
    gl = (jnp.dot(wlin_ref[...], jk, preferred_element_type=_f32)
          + blin_ref[...])
    f1 = jnp.sum(gl * wfc1_ref[...], axis=0, keepdims=True) + bfc1_ref[...]
    f1 = jnp.where(f1 >= 0, f1, 0.01 * f1)
    val = jnp.sum(f1 * wfc2_ref[...]) + bfc2_ref[0, 0]
    o_ref[...] = val.reshape(1, 1)


_readout = pl.pallas_call(
    _readout_body, out_shape=jax.ShapeDtypeStruct((1, 1), _f32))


def _padm(w, r, c):
    return jnp.zeros((r, c), _f32).at[: w.shape[0], : w.shape[1]].set(w)


def _padb(b, n=D):
    return jnp.zeros((n, 1), _f32).at[: b.shape[0], 0].set(b)


def _pack_lstm(wih, whh, bih, bhh):
    wi = jnp.zeros((G4, D), _f32)
    wh = jnp.zeros((G4, D), _f32)
    bias = jnp.zeros((G4, 1), _f32)
    bsum = bih + bhh
    for k in range(4):
        wi = wi.at[k * 8 : k * 8 + 7, : wih.shape[1]].set(wih[k * 7 : (k + 1) * 7])
        wh = wh.at[k * 8 : k * 8 + 7, : whh.shape[1]].set(whh[k * 7 : (k + 1) * 7])
        bias = bias.at[k * 8 : k * 8 + 7, 0].set(bsum[k * 7 : (k + 1) * 7])
    return wi, wh, bias


def kernel(x, edge_index, W0a, b0a, W0b, b0b, W1a, b1a, W1b, b1b, W2a, b2a,
           W2b, b2b, lstm_Wih_f, lstm_Whh_f, lstm_bih_f, lstm_bhh_f,
           lstm_Wih_b, lstm_Whh_b, lstm_bih_b, lstm_bhh_b,
           Watt, batt, Wlin, blin, Wfc1, bfc1, Wfc2, bfc2):
    er = edge_index.reshape(2, NW, NCHT, CH)

    w0a_p = _padm(W0a, D, F)
    w0b_p = _padm(W0b, D, D)
    w1a_p = _padm(W1a, D, D)
    w1b_p = _padm(W1b, D, D)
    w2a_p = _padm(W2a, D, D)
    w2b_p = _padm(W2b, D, D)
    b0a_p, b0b_p = _padb(b0a), _padb(b0b)
    b1a_p, b1b_p = _padb(b1a), _padb(b1b)
    b2a_p, b2b_p = _padb(b2a), _padb(b2b)
    wih_f, whh_f, bias_f = _pack_lstm(lstm_Wih_f, lstm_Whh_f,
                                      lstm_bih_f, lstm_bhh_f)
    wih_b, whh_b, bias_b = _pack_lstm(lstm_Wih_b, lstm_Whh_b,
                                      lstm_bih_b, lstm_bhh_b)
    wattf = _padb(Watt[0, :7])
    wattb = _padb(Watt[0, 7:14])
    batt_p = batt.reshape(1, 1)
    wlin_p = _padm(Wlin, D, D)
    blin_p = _padb(blin)
    wfc1_p = _padb(Wfc1[0])
    bfc1_p = bfc1.reshape(1, 1)
    wfc2_p = jnp.zeros((1, NPAD), _f32).at[0, :N].set(Wfc2[0])
    bfc2_p = bfc2.reshape(1, 1)

    agg = _make_agg()
    u0T = jnp.pad(_proj(x, w0a_p), ((0, 0), (0, NPAD - N)))
    p0 = agg(u0T, er)
    h1T, v1T = _gin(u0T, p0[0], p0[1], b0a_p, w0b_p, b0b_p, w1a_p)
    p1 = agg(v1T, er)
    h2T, v2T = _gin(v1T, p1[0], p1[1], b1a_p, w1b_p, b1b_p, w2a_p)
    p2 = agg(v2T, er)
    res = _readout(v2T, p2[0], p2[1], b2a_p, w2b_p, b2b_p, h1T, h2T,
                   wih_f, whh_f, bias_f, wih_b, whh_b, bias_b,
                   wattf, wattb, batt_p, wlin_p, blin_p, wfc1_p, bfc1_p,
                   wfc2_p, bfc2_p)
    return res[0]
